# baseline (speedup 1.0000x reference)
import functools

import jax
import jax.numpy as jnp
from jax import lax
from jax.experimental import pallas as pl
from jax.experimental.pallas import tpu as pltpu
from jax.experimental.pallas import tpu_sc as plsc

NUM_EMB = 1000000
DIM = 32
B = 16384

_NC = 2
_NS = 16
_NW = _NC * _NS
_BPW = B // _NW
_CHUNK = 128
_NCH = _BPW // _CHUNK

_mesh = plsc.VectorSubcoreMesh(core_axis_name="c", subcore_axis_name="s")


@functools.partial(
    pl.kernel,
    mesh=_mesh,
    compiler_params=pltpu.CompilerParams(use_tc_tiling_on_sc=False),
    out_type=jax.ShapeDtypeStruct((2, B, DIM), jnp.float32),
    scratch_types=[
        pltpu.VMEM((_NCH, _CHUNK), jnp.int32),
        pltpu.VMEM((_BPW, DIM), jnp.float32),
        pltpu.VMEM((_BPW, DIM), jnp.float32),
        pltpu.SemaphoreType.DMA,
    ],
)
def _emb_lookup(idx_hbm, wr_hbm, wi_hbm, out_hbm, idx_v, rows_r, rows_i, sem):
    wid = lax.axis_index("s") * _NC + lax.axis_index("c")
    base = wid * _BPW
    pltpu.sync_copy(idx_hbm.at[wid], idx_v)
    copies = []
    for j in range(_NCH):
        dst = pl.ds(j * _CHUNK, _CHUNK)
        copies.append(pltpu.async_copy(wr_hbm.at[idx_v.at[j]], rows_r.at[dst], sem))
        copies.append(pltpu.async_copy(wi_hbm.at[idx_v.at[j]], rows_i.at[dst], sem))
    for c in copies:
        c.wait()
    pltpu.sync_copy(rows_r, out_hbm.at[0, pl.ds(base, _BPW)])
    pltpu.sync_copy(rows_i, out_hbm.at[1, pl.ds(base, _BPW)])


@jax.jit
def kernel(index, W_real, W_img):
    idx = index.astype(jnp.int32).reshape(_NW, _NCH, _CHUNK)
    return _emb_lookup(idx, W_real, W_img)
